# baseline (speedup 1.0000x reference)
import functools

import jax
import jax.numpy as jnp
from jax import lax
from jax.experimental import pallas as pl
from jax.experimental.pallas import tpu as pltpu
from jax.experimental.pallas import tpu_sc as plsc

N = 16384
NCAT = 26
NNUM = 13
NCOL = NCAT + NNUM
VOCAB = 1000
C = 128
NW = 32
RPW = N // NW
RC = 4
IPC = RC * NCAT
NPC = RC * NNUM
NCH = RPW // RC
NSLOT = 2
LANES = 16

_mesh = plsc.VectorSubcoreMesh(core_axis_name="c", subcore_axis_name="s")


@functools.partial(
    pl.kernel,
    mesh=_mesh,
    out_type=jax.ShapeDtypeStruct((N * NCOL, C), jnp.float32),
    scratch_types=[
        pltpu.VMEM((NCH, IPC), jnp.int32),
        pltpu.VMEM((NCH, IPC), jnp.int32),
        pltpu.VMEM((NCH, NPC), jnp.int32),
        pltpu.VMEM((NNUM, NCH * LANES), jnp.float32),
        pltpu.VMEM((NNUM, C), jnp.float32),
        pltpu.VMEM((NNUM, C), jnp.float32),
        pltpu.VMEM((NSLOT, IPC, C), jnp.float32),
        pltpu.VMEM((NSLOT, NPC, C), jnp.float32),
        pltpu.SemaphoreType.DMA((NSLOT,)),
        pltpu.SemaphoreType.DMA((NSLOT,)),
    ],
)
def _encode(table_hbm, idx_hbm, dstc_hbm, dstn_hbm, fnum_hbm, w_hbm, b_hbm,
            out_hbm, idx_v, dstc_v, dstn_v, fnum_v, w_v, b_v, gbuf, nbuf,
            gsem, wsem):
    wid = lax.axis_index("s") * 2 + lax.axis_index("c")
    pltpu.sync_copy(idx_hbm.at[wid], idx_v)
    pltpu.sync_copy(dstc_hbm.at[wid], dstc_v)
    pltpu.sync_copy(dstn_hbm.at[wid], dstn_v)
    pltpu.sync_copy(fnum_hbm.at[wid], fnum_v)
    pltpu.sync_copy(w_hbm, w_v)
    pltpu.sync_copy(b_hbm, b_v)

    pltpu.async_copy(table_hbm.at[idx_v.at[0]], gbuf.at[0], gsem.at[0])

    def chunk(c, carry):
        s = c % NSLOT

        def jbody(j, carry2):
            v16 = fnum_v[j, pl.ds(c * LANES, LANES)]
            for r in range(RC):
                vb = jnp.full((LANES,), v16[r], dtype=jnp.float32)
                for k in range(C // LANES):
                    sl = pl.ds(k * LANES, LANES)
                    nbuf[s, r * NNUM + j, sl] = vb * w_v[j, sl] + b_v[j, sl]
            return carry2

        lax.fori_loop(0, NNUM, jbody, 0)

        @pl.when(c >= 1)
        def _():
            sp = (c + 1) % NSLOT
            pltpu.make_async_copy(gbuf.at[sp], out_hbm.at[dstc_v.at[c - 1]],
                                  wsem.at[sp]).wait()
            pltpu.make_async_copy(nbuf.at[sp], out_hbm.at[dstn_v.at[c - 1]],
                                  wsem.at[sp]).wait()

        @pl.when(c < NCH - 1)
        def _():
            sn = (c + 1) % NSLOT
            pltpu.async_copy(table_hbm.at[idx_v.at[c + 1]], gbuf.at[sn],
                             gsem.at[sn])

        pltpu.make_async_copy(table_hbm.at[idx_v.at[c]], gbuf.at[s],
                              gsem.at[s]).wait()

        pltpu.async_copy(gbuf.at[s], out_hbm.at[dstc_v.at[c]], wsem.at[s])
        pltpu.async_copy(nbuf.at[s], out_hbm.at[dstn_v.at[c]], wsem.at[s])
        return carry

    lax.fori_loop(0, NCH, chunk, 0)
    sl = (NCH - 1) % NSLOT
    pltpu.make_async_copy(gbuf.at[sl], out_hbm.at[dstc_v.at[NCH - 1]],
                          wsem.at[sl]).wait()
    pltpu.make_async_copy(nbuf.at[sl], out_hbm.at[dstn_v.at[NCH - 1]],
                          wsem.at[sl]).wait()


def kernel(feat_cat, feat_num, emb_tables, lin_weight, lin_bias, num_mean, num_std):
    table = emb_tables.reshape(NCAT * VOCAB, C)
    offs = jnp.arange(NCAT, dtype=jnp.int32) * VOCAB
    idx = (feat_cat.astype(jnp.int32) + offs[None, :]).reshape(NW, NCH, IPC)
    n_grid = jnp.arange(N, dtype=jnp.int32).reshape(NW, NCH, RC)
    dstc = (n_grid[..., None] * NCOL
            + jnp.arange(NCAT, dtype=jnp.int32)).reshape(NW, NCH, IPC)
    dstn = (n_grid[..., None] * NCOL + NCAT
            + jnp.arange(NNUM, dtype=jnp.int32)).reshape(NW, NCH, NPC)
    fnum = feat_num.reshape(NW, NCH, RC, NNUM).transpose(0, 3, 1, 2)
    fnum = jnp.pad(fnum, ((0, 0), (0, 0), (0, 0), (0, LANES - RC)))
    fnum = fnum.reshape(NW, NNUM, NCH * LANES)
    inv = 1.0 / num_std
    w_eff = lin_weight * inv[:, None]
    b_eff = lin_bias - (num_mean * inv)[:, None] * lin_weight
    out = _encode(table, idx, dstc, dstn, fnum, w_eff, b_eff)
    return out.reshape(N, NCOL, C)


# speedup vs baseline: 2.7594x; 2.7594x over previous
import functools

import jax
import jax.numpy as jnp
from jax import lax
from jax.experimental import pallas as pl
from jax.experimental.pallas import tpu as pltpu
from jax.experimental.pallas import tpu_sc as plsc

N = 16384
NCAT = 26
NNUM = 13
NCOL = NCAT + NNUM
VOCAB = 1000
C = 128
NW = 32
RPW = N // NW
HR = RPW // 2
QR = RPW // 4
NCH = 2 * NCAT
IPD = 128
GPC = HR // IPD
NSLOT = 2
LANES = 16

_mesh = plsc.VectorSubcoreMesh(core_axis_name="c", subcore_axis_name="s")


@functools.partial(
    pl.kernel,
    mesh=_mesh,
    out_type=jax.ShapeDtypeStruct((N * NCOL, C), jnp.float32),
    scratch_types=[
        pltpu.VMEM((NCAT * 4, IPD), jnp.int32),
        pltpu.VMEM((NNUM, RPW), jnp.float32),
        pltpu.VMEM((NNUM, C), jnp.float32),
        pltpu.VMEM((NNUM, C), jnp.float32),
        pltpu.VMEM((NSLOT, HR, C), jnp.float32),
        pltpu.VMEM((NSLOT, QR, C), jnp.float32),
        pltpu.SemaphoreType.DMA((NSLOT,)),
        pltpu.SemaphoreType.DMA((NSLOT,)),
    ],
)
def _encode(table_hbm, idx_hbm, fnum_hbm, w_hbm, b_hbm,
            out_hbm, idx_v, fnum_v, w_v, b_v, gbuf, nbuf, gsem, wsem):
    wid = lax.axis_index("s") * 2 + lax.axis_index("c")
    pltpu.sync_copy(idx_hbm.at[wid], idx_v)
    pltpu.sync_copy(fnum_hbm.at[wid], fnum_v)
    pltpu.sync_copy(w_hbm, w_v)
    pltpu.sync_copy(b_hbm, b_v)
    wbase = wid * RPW

    def cat_dst(t):
        return (t // 2) * N + wbase + (t % 2) * HR

    def num_dst(t):
        return (NCAT + t // 4) * N + wbase + (t % 4) * QR

    def fire_gathers(t, s):
        for q in range(GPC):
            pltpu.async_copy(table_hbm.at[idx_v.at[t * GPC + q]],
                             gbuf.at[s, pl.ds(q * IPD, IPD)], gsem.at[s])

    fire_gathers(0, 0)

    def chunk(t, carry):
        s = t % NSLOT
        jn = t // 4
        qbase = (t % 4) * QR

        def ibody(i, carry2):
            v16 = fnum_v[jn, pl.ds(qbase + i * LANES, LANES)]
            for r in range(LANES):
                vb = jnp.full((LANES,), v16[r], dtype=jnp.float32)
                for k in range(C // LANES):
                    sl = pl.ds(k * LANES, LANES)
                    nbuf[s, i * LANES + r, sl] = vb * w_v[jn, sl] + b_v[jn, sl]
            return carry2

        lax.fori_loop(0, QR // LANES, ibody, 0)

        @pl.when(t >= 1)
        def _():
            sp = (t + 1) % NSLOT
            pltpu.make_async_copy(
                gbuf.at[sp], out_hbm.at[pl.ds(cat_dst(t - 1), HR)],
                wsem.at[sp]).wait()
            pltpu.make_async_copy(
                nbuf.at[sp], out_hbm.at[pl.ds(num_dst(t - 1), QR)],
                wsem.at[sp]).wait()

        @pl.when(t < NCH - 1)
        def _():
            fire_gathers(t + 1, (t + 1) % NSLOT)

        for q in range(GPC):
            pltpu.make_async_copy(table_hbm.at[idx_v.at[t * GPC + q]],
                                  gbuf.at[s, pl.ds(q * IPD, IPD)],
                                  gsem.at[s]).wait()

        pltpu.async_copy(gbuf.at[s], out_hbm.at[pl.ds(cat_dst(t), HR)],
                         wsem.at[s])
        pltpu.async_copy(nbuf.at[s], out_hbm.at[pl.ds(num_dst(t), QR)],
                         wsem.at[s])
        return carry

    lax.fori_loop(0, NCH, chunk, 0)
    sl = (NCH - 1) % NSLOT
    pltpu.make_async_copy(gbuf.at[sl], out_hbm.at[pl.ds(cat_dst(NCH - 1), HR)],
                          wsem.at[sl]).wait()
    pltpu.make_async_copy(nbuf.at[sl], out_hbm.at[pl.ds(num_dst(NCH - 1), QR)],
                          wsem.at[sl]).wait()


def kernel(feat_cat, feat_num, emb_tables, lin_weight, lin_bias, num_mean, num_std):
    table = emb_tables.reshape(NCAT * VOCAB, C)
    offs = jnp.arange(NCAT, dtype=jnp.int32) * VOCAB
    idx = (feat_cat.astype(jnp.int32) + offs[None, :]).T
    idx = idx.reshape(NCAT, NW, 4, IPD).transpose(1, 0, 2, 3)
    idx = idx.reshape(NW, NCAT * 4, IPD)
    fnum = feat_num.T.reshape(NNUM, NW, RPW).transpose(1, 0, 2)
    inv = 1.0 / num_std
    w_eff = lin_weight * inv[:, None]
    b_eff = lin_bias - (num_mean * inv)[:, None] * lin_weight
    out = _encode(table, idx, fnum, w_eff, b_eff)
    return out.reshape(NCOL, N, C).transpose(1, 0, 2)


# speedup vs baseline: 2.9318x; 1.0625x over previous
import functools

import jax
import jax.numpy as jnp
from jax import lax
from jax.experimental import pallas as pl
from jax.experimental.pallas import tpu as pltpu
from jax.experimental.pallas import tpu_sc as plsc

N = 16384
NCAT = 26
NNUM = 13
NCOL = NCAT + NNUM
VOCAB = 1000
C = 128
NW = 32
RPW = N // NW
HR = RPW // 2
NCH = 2 * NCAT
IPD = 128
GPC = HR // IPD
NSLOT = 3
LANES = 16
NBLK = 1024

_mesh = plsc.VectorSubcoreMesh(core_axis_name="c", subcore_axis_name="s")


@functools.partial(
    pl.kernel,
    mesh=_mesh,
    out_type=jax.ShapeDtypeStruct((N * NCOL, C), jnp.float32),
    scratch_types=[
        pltpu.VMEM((NCAT * 4, IPD), jnp.int32),
        pltpu.VMEM((NSLOT, HR, C), jnp.float32),
        pltpu.SemaphoreType.DMA((NSLOT,)),
        pltpu.SemaphoreType.DMA((NSLOT,)),
    ],
)
def _encode_cat(table_hbm, idx_hbm, out_hbm, idx_v, gbuf, gsem, wsem):
    wid = lax.axis_index("s") * 2 + lax.axis_index("c")
    pltpu.sync_copy(idx_hbm.at[wid], idx_v)
    wbase = wid * RPW

    def cat_dst(t):
        return (t // 2) * N + wbase + (t % 2) * HR

    def fire_gathers(t, s):
        for q in range(GPC):
            pltpu.async_copy(table_hbm.at[idx_v.at[t * GPC + q]],
                             gbuf.at[s, pl.ds(q * IPD, IPD)], gsem.at[s])

    fire_gathers(0, 0)
    fire_gathers(1, 1)

    def chunk(t, carry):
        s = t % NSLOT

        @pl.when(t >= 1)
        def _():
            sp = (t + 2) % NSLOT
            pltpu.make_async_copy(
                gbuf.at[sp], out_hbm.at[pl.ds(cat_dst(t - 1), HR)],
                wsem.at[sp]).wait()

        @pl.when(t < NCH - 2)
        def _():
            fire_gathers(t + 2, (t + 2) % NSLOT)

        for q in range(GPC):
            pltpu.make_async_copy(table_hbm.at[idx_v.at[t * GPC + q]],
                                  gbuf.at[s, pl.ds(q * IPD, IPD)],
                                  gsem.at[s]).wait()

        pltpu.async_copy(gbuf.at[s], out_hbm.at[pl.ds(cat_dst(t), HR)],
                         wsem.at[s])
        return carry

    lax.fori_loop(0, NCH, chunk, 0)
    sl = (NCH - 1) % NSLOT
    pltpu.make_async_copy(gbuf.at[sl], out_hbm.at[pl.ds(cat_dst(NCH - 1), HR)],
                          wsem.at[sl]).wait()


def _num_body(buf_ref, fn_ref, w_ref, b_ref, out_ref):
    del buf_ref
    fn = fn_ref[...]
    out_ref[...] = fn[0, 0, :, None] * w_ref[...][0] + b_ref[...][0]


_num_fill = pl.pallas_call(
    _num_body,
    grid=(NNUM * (N // NBLK),),
    in_specs=[
        pl.BlockSpec(memory_space=pl.ANY),
        pl.BlockSpec((1, 1, NBLK),
                     lambda i: (i // (N // NBLK) * (N // NBLK)
                                + i % (N // NBLK), 0, 0)),
        pl.BlockSpec((1, 1, C), lambda i: (i // (N // NBLK), 0, 0)),
        pl.BlockSpec((1, 1, C), lambda i: (i // (N // NBLK), 0, 0)),
    ],
    out_specs=pl.BlockSpec((NBLK, C), lambda i: (NCAT * (N // NBLK) + i, 0)),
    out_shape=jax.ShapeDtypeStruct((N * NCOL, C), jnp.float32),
    input_output_aliases={0: 0},
)


def kernel(feat_cat, feat_num, emb_tables, lin_weight, lin_bias, num_mean, num_std):
    table = emb_tables.reshape(NCAT * VOCAB, C)
    offs = jnp.arange(NCAT, dtype=jnp.int32) * VOCAB
    idx = (feat_cat.astype(jnp.int32) + offs[None, :]).T
    idx = idx.reshape(NCAT, NW, 4, IPD).transpose(1, 0, 2, 3)
    idx = idx.reshape(NW, NCAT * 4, IPD)
    inv = 1.0 / num_std
    w_eff = lin_weight * inv[:, None]
    b_eff = lin_bias - (num_mean * inv)[:, None] * lin_weight
    out = _encode_cat(table, idx)
    fnum_blk = feat_num.T.reshape(NNUM * (N // NBLK), 1, NBLK)
    out = _num_fill(out, fnum_blk, w_eff[:, None, :], b_eff[:, None, :])
    return out.reshape(NCOL, N, C).transpose(1, 0, 2)
